# baseline (speedup 1.0000x reference)
import functools

import jax
import jax.numpy as jnp
from jax import lax
from jax.experimental import pallas as pl
from jax.experimental.pallas import tpu as pltpu
from jax.experimental.pallas import tpu_sc as plsc

N = 10000
E = 320000
F_IN = 128
H = 256
HH = H // 2

NC = 2
NS = 16
NW = NC * NS

CH = 80
RPT = N // NS

_sc_mesh = plsc.VectorSubcoreMesh(core_axis_name="c", subcore_axis_name="s")


@functools.partial(
    pl.kernel,
    mesh=_sc_mesh,
    out_type=jax.ShapeDtypeStruct((NW, N), jnp.float32),
    scratch_types=[
        pltpu.VMEM((N,), jnp.float32),
        pltpu.VMEM((E // NW,), jnp.int32),
        pltpu.SemaphoreType.DMA,
    ],
    compiler_params=pltpu.CompilerParams(needs_layout_passes=False),
)
def _deg_kernel(dst_hbm, out_hbm, acc, dbuf, dsem):
    c = lax.axis_index("c")
    s = lax.axis_index("s")
    wid = s * NC + c
    ep = E // NW

    din = pltpu.async_copy(dst_hbm.at[pl.ds(wid * ep, ep)], dbuf, dsem)

    z16 = jnp.zeros((16,), jnp.float32)

    def zinit(i, _):
        acc[pl.ds(i * 16, 16)] = z16
        return 0

    lax.fori_loop(0, N // 16, zinit, 0)
    din.wait()

    ones = jnp.full((16,), 1.0, jnp.float32)

    def body(i, _):
        for j in range(CH // 16):
            idx = dbuf[pl.ds(i * CH + j * 16, 16)]
            plsc.addupdate_scatter(acc, [idx], ones)
        return 0

    lax.fori_loop(0, ep // CH, body, 0)
    pltpu.sync_copy(acc, out_hbm.at[wid])


NCHUNK = E // NS // CH
NPH = 2
PCH = NCHUNK // NPH
NBUF = 3


@functools.partial(
    pl.kernel,
    mesh=_sc_mesh,
    out_type=jax.ShapeDtypeStruct((2 * N, HH), jnp.float32),
    scratch_types=[
        pltpu.VMEM((PCH, CH), jnp.int32),
        pltpu.VMEM((PCH, CH), jnp.int32),
        [pltpu.VMEM((CH, HH), jnp.float32)] * NBUF,
        [pltpu.SemaphoreType.DMA] * NBUF,
        [pltpu.SemaphoreType.DMA] * NBUF,
        pltpu.VMEM_SHARED((N, HH), jnp.float32),
        pltpu.SemaphoreType.DMA,
    ],
    compiler_params=pltpu.CompilerParams(use_tc_tiling_on_sc=False),
)
def _prop_kernel(y_hbm, srcoff_hbm, dst_hbm, z_hbm, sidx, didx, bufs, gsems,
                 ssems, zacc, isem):
    c = lax.axis_index("c")
    s = lax.axis_index("s")
    rbase = s * RPT

    pltpu.sync_copy(
        y_hbm.at[pl.ds(c * N + rbase, RPT)], zacc.at[pl.ds(rbase, RPT)]
    )
    plsc.subcore_barrier()

    def g_start(i, b):
        pltpu.async_copy(y_hbm.at[sidx.at[i]], bufs[b], gsems[b])

    def g_wait(b):
        pltpu.make_async_copy(y_hbm.at[sidx.at[0]], bufs[b], gsems[b]).wait()

    def s_start(i, b):
        pltpu.async_copy(bufs[b], zacc.at[didx.at[i]], ssems[b], add=True)

    def s_wait(b):
        pltpu.make_async_copy(bufs[b], zacc.at[didx.at[0]], ssems[b]).wait()

    def phase(p, _):
        iin = pltpu.async_copy(srcoff_hbm.at[c, s, p], sidx, isem)
        pltpu.sync_copy(dst_hbm.at[s, p], didx)
        iin.wait()
        for j in range(NBUF - 1):
            g_start(j, j)

        def step(i, _):
            for b in range(NBUF):
                @pl.when(lax.rem(i, NBUF) == b)
                def _(b=b):
                    nb = (b + NBUF - 1) % NBUF
                    g_wait(b)
                    s_start(i, b)

                    @pl.when(i >= 1)
                    def _():
                        s_wait(nb)

                    @pl.when(i + NBUF - 1 < PCH)
                    def _():
                        g_start(i + NBUF - 1, nb)

            return 0

        lax.fori_loop(0, PCH, step, 0)
        s_wait((PCH - 1) % NBUF)
        return 0

    lax.fori_loop(0, NPH, phase, 0)
    plsc.subcore_barrier()

    pltpu.sync_copy(
        zacc.at[pl.ds(rbase, RPT)], z_hbm.at[pl.ds(c * N + rbase, RPT)]
    )


def _mm1_body(x_ref, w_ref, degt_ref, y_ref):
    deg = jnp.sum(degt_ref[...], axis=1, keepdims=True) + 1.0
    dinv = lax.rsqrt(deg)
    xw = jnp.dot(x_ref[...].astype(jnp.bfloat16),
                 w_ref[...].astype(jnp.bfloat16),
                 preferred_element_type=jnp.float32)
    y_ref[...] = xw * dinv


_BLK = 2000
_NB = N // _BLK


def _mm1(x, w1, degt):
    return pl.pallas_call(
        _mm1_body,
        grid=(_NB, 2),
        in_specs=[
            pl.BlockSpec((_BLK, F_IN), lambda i, j: (i, 0)),
            pl.BlockSpec((F_IN, HH), lambda i, j: (0, j)),
            pl.BlockSpec((_BLK, NW), lambda i, j: (i, 0)),
        ],
        out_specs=pl.BlockSpec((_BLK, HH), lambda i, j: (j * _NB + i, 0)),
        out_shape=jax.ShapeDtypeStruct((2 * N, HH), jnp.float32),
    )(x, w1, degt)


def _mm2_body(za_ref, zb_ref, wa_ref, wb_ref, ba_ref, bb_ref, degt_ref, y_ref):
    deg = jnp.sum(degt_ref[...], axis=1, keepdims=True) + 1.0
    dinv = lax.rsqrt(deg)
    ha = jnp.maximum(za_ref[...] * dinv + ba_ref[0], 0.0)
    hb = jnp.maximum(zb_ref[...] * dinv + bb_ref[0], 0.0)
    acc = jnp.dot(ha.astype(jnp.bfloat16), wa_ref[...].astype(jnp.bfloat16),
                  preferred_element_type=jnp.float32)
    acc = acc + jnp.dot(hb.astype(jnp.bfloat16),
                        wb_ref[...].astype(jnp.bfloat16),
                        preferred_element_type=jnp.float32)
    y_ref[...] = acc * dinv


def _mm2(z1, w2, b1r, degt):
    return pl.pallas_call(
        _mm2_body,
        grid=(_NB, 2),
        in_specs=[
            pl.BlockSpec((_BLK, HH), lambda i, j: (i, 0)),
            pl.BlockSpec((_BLK, HH), lambda i, j: (_NB + i, 0)),
            pl.BlockSpec((HH, HH), lambda i, j: (0, j)),
            pl.BlockSpec((HH, HH), lambda i, j: (1, j)),
            pl.BlockSpec((1, 1, HH), lambda i, j: (0, 0, 0)),
            pl.BlockSpec((1, 1, HH), lambda i, j: (1, 0, 0)),
            pl.BlockSpec((_BLK, NW), lambda i, j: (i, 0)),
        ],
        out_specs=pl.BlockSpec((_BLK, HH), lambda i, j: (j * _NB + i, 0)),
        out_shape=jax.ShapeDtypeStruct((2 * N, HH), jnp.float32),
    )(z1, z1, w2, w2, b1r, b1r, degt)


def _mm3_body(za_ref, zb_ref, wa_ref, wb_ref, ba_ref, bb_ref, bl_ref, degt_ref,
              o_ref):
    deg = jnp.sum(degt_ref[...], axis=1, keepdims=True) + 1.0
    dinv = lax.rsqrt(deg)
    ha = jnp.maximum(za_ref[...] * dinv + ba_ref[0], 0.0)
    hb = jnp.maximum(zb_ref[...] * dinv + bb_ref[0], 0.0)
    acc = jnp.dot(ha.astype(jnp.bfloat16), wa_ref[...].astype(jnp.bfloat16),
                  preferred_element_type=jnp.float32)
    acc = acc + jnp.dot(hb.astype(jnp.bfloat16),
                        wb_ref[...].astype(jnp.bfloat16),
                        preferred_element_type=jnp.float32)
    o_ref[...] = acc + bl_ref[...]


def _mm3(z2, wl, b2r, bl, degt):
    return pl.pallas_call(
        _mm3_body,
        grid=(_NB,),
        in_specs=[
            pl.BlockSpec((_BLK, HH), lambda i: (i, 0)),
            pl.BlockSpec((_BLK, HH), lambda i: (_NB + i, 0)),
            pl.BlockSpec((HH, 1), lambda i: (0, 0)),
            pl.BlockSpec((HH, 1), lambda i: (1, 0)),
            pl.BlockSpec((1, 1, HH), lambda i: (0, 0, 0)),
            pl.BlockSpec((1, 1, HH), lambda i: (1, 0, 0)),
            pl.BlockSpec((1, 1), lambda i: (0, 0)),
            pl.BlockSpec((_BLK, NW), lambda i: (i, 0)),
        ],
        out_specs=pl.BlockSpec((_BLK, 1), lambda i: (i, 0)),
        out_shape=jax.ShapeDtypeStruct((N, 1), jnp.float32),
    )(z2, z2, wl, wl, b2r, b2r, bl.reshape(1, 1), degt)


@jax.jit
def kernel(x, edge_index, batch, W1, b1, W2, b2, Wl, bl):
    src = edge_index[0]
    dst = edge_index[1]
    srcoff = jnp.stack([src, src + N]).reshape(2, NS, NPH, PCH, CH)
    dst4 = dst.reshape(NS, NPH, PCH, CH)

    degp = _deg_kernel(dst)
    degt = degp.T

    y1 = _mm1(x, W1, degt)
    z1 = _prop_kernel(y1, srcoff, dst4)
    y2 = _mm2(z1, W2, b1.reshape(2, 1, HH), degt)
    z2 = _prop_kernel(y2, srcoff, dst4)
    out = _mm3(z2, Wl, b2.reshape(2, 1, HH), bl, degt)
    return out


# speedup vs baseline: 1.0358x; 1.0358x over previous
import functools

import jax
import jax.numpy as jnp
from jax import lax
from jax.experimental import pallas as pl
from jax.experimental.pallas import tpu as pltpu
from jax.experimental.pallas import tpu_sc as plsc

N = 10000
E = 320000
F_IN = 128
H = 256
HH = H // 2

NC = 2
NS = 16
NW = NC * NS

CH = 80
RPT = N // NS

_sc_mesh = plsc.VectorSubcoreMesh(core_axis_name="c", subcore_axis_name="s")


NROW = E // CH
RPW = NROW // NW


@functools.partial(
    pl.kernel,
    mesh=_sc_mesh,
    out_type=[
        jax.ShapeDtypeStruct((NW, N), jnp.float32),
        jax.ShapeDtypeStruct((3, NROW, CH), jnp.int32),
    ],
    scratch_types=[
        pltpu.VMEM((N,), jnp.float32),
        pltpu.VMEM((RPW, CH), jnp.int32),
        pltpu.VMEM((RPW, CH), jnp.int32),
        pltpu.SemaphoreType.DMA,
        pltpu.SemaphoreType.DMA,
    ],
    compiler_params=pltpu.CompilerParams(
        needs_layout_passes=False, use_tc_tiling_on_sc=False
    ),
)
def _deg_kernel(ei_hbm, out_hbm, eidx_hbm, acc, dbuf, sbuf, dsem, ssem):
    c = lax.axis_index("c")
    s = lax.axis_index("s")
    wid = s * NC + c
    rlo = wid * RPW

    din = pltpu.async_copy(ei_hbm.at[1, pl.ds(rlo, RPW)], dbuf, dsem)
    sin = pltpu.async_copy(ei_hbm.at[0, pl.ds(rlo, RPW)], sbuf, ssem)

    z16 = jnp.zeros((16,), jnp.float32)

    def zinit(i, _):
        acc[pl.ds(i * 16, 16)] = z16
        return 0

    lax.fori_loop(0, N // 16, zinit, 0)
    din.wait()
    dout = pltpu.async_copy(dbuf, eidx_hbm.at[2, pl.ds(rlo, RPW)], dsem)
    sin.wait()
    sout = pltpu.async_copy(sbuf, eidx_hbm.at[0, pl.ds(rlo, RPW)], ssem)

    ones = jnp.full((16,), 1.0, jnp.float32)

    def body(i, _):
        for j in range(CH // 16):
            idx = dbuf[i, pl.ds(j * 16, 16)]
            plsc.addupdate_scatter(acc, [idx], ones)
        return 0

    lax.fori_loop(0, RPW, body, 0)
    pltpu.sync_copy(acc, out_hbm.at[wid])

    sout.wait()

    def addn(i, _):
        for j in range(CH // 16):
            sbuf[i, pl.ds(j * 16, 16)] = sbuf[i, pl.ds(j * 16, 16)] + N
        return 0

    lax.fori_loop(0, RPW, addn, 0)
    pltpu.sync_copy(sbuf, eidx_hbm.at[1, pl.ds(rlo, RPW)])
    dout.wait()


NCHUNK = E // NS // CH
NPH = 2
PCH = NCHUNK // NPH
NBUF = 3


@functools.partial(
    pl.kernel,
    mesh=_sc_mesh,
    out_type=jax.ShapeDtypeStruct((2 * N, HH), jnp.float32),
    scratch_types=[
        pltpu.VMEM((PCH, CH), jnp.int32),
        pltpu.VMEM((PCH, CH), jnp.int32),
        [pltpu.VMEM((CH, HH), jnp.float32)] * NBUF,
        [pltpu.SemaphoreType.DMA] * NBUF,
        [pltpu.SemaphoreType.DMA] * NBUF,
        pltpu.VMEM_SHARED((N, HH), jnp.float32),
        pltpu.SemaphoreType.DMA,
    ],
    compiler_params=pltpu.CompilerParams(use_tc_tiling_on_sc=False),
)
def _prop_kernel(y_hbm, eidx_hbm, z_hbm, sidx, didx, bufs, gsems,
                 ssems, zacc, isem):
    c = lax.axis_index("c")
    s = lax.axis_index("s")
    rbase = s * RPT

    pltpu.sync_copy(
        y_hbm.at[pl.ds(c * N + rbase, RPT)], zacc.at[pl.ds(rbase, RPT)]
    )
    plsc.subcore_barrier()

    def g_start(i, b):
        pltpu.async_copy(y_hbm.at[sidx.at[i]], bufs[b], gsems[b])

    def g_wait(b):
        pltpu.make_async_copy(y_hbm.at[sidx.at[0]], bufs[b], gsems[b]).wait()

    def s_start(i, b):
        pltpu.async_copy(bufs[b], zacc.at[didx.at[i]], ssems[b], add=True)

    def s_wait(b):
        pltpu.make_async_copy(bufs[b], zacc.at[didx.at[0]], ssems[b]).wait()

    def phase(p, _):
        roff = s * (NPH * PCH) + p * PCH
        iin = pltpu.async_copy(eidx_hbm.at[c, pl.ds(roff, PCH)], sidx, isem)
        pltpu.sync_copy(eidx_hbm.at[2, pl.ds(roff, PCH)], didx)
        iin.wait()
        for j in range(NBUF - 1):
            g_start(j, j)

        def step(i, _):
            for b in range(NBUF):
                @pl.when(lax.rem(i, NBUF) == b)
                def _(b=b):
                    nb = (b + NBUF - 1) % NBUF
                    g_wait(b)
                    s_start(i, b)

                    @pl.when(i >= 1)
                    def _():
                        s_wait(nb)

                    @pl.when(i + NBUF - 1 < PCH)
                    def _():
                        g_start(i + NBUF - 1, nb)

            return 0

        lax.fori_loop(0, PCH, step, 0)
        s_wait((PCH - 1) % NBUF)
        return 0

    lax.fori_loop(0, NPH, phase, 0)
    plsc.subcore_barrier()

    pltpu.sync_copy(
        zacc.at[pl.ds(rbase, RPT)], z_hbm.at[pl.ds(c * N + rbase, RPT)]
    )


def _mm1_body(x_ref, w_ref, degt_ref, y_ref):
    deg = jnp.sum(degt_ref[...], axis=1, keepdims=True) + 1.0
    dinv = lax.rsqrt(deg)
    xw = jnp.dot(x_ref[...], w_ref[...], preferred_element_type=jnp.float32)
    y_ref[...] = xw * dinv


_BLK = 2000
_NB = N // _BLK


def _mm1(x, w1, degt):
    return pl.pallas_call(
        _mm1_body,
        grid=(_NB, 2),
        in_specs=[
            pl.BlockSpec((_BLK, F_IN), lambda i, j: (i, 0)),
            pl.BlockSpec((F_IN, HH), lambda i, j: (0, j)),
            pl.BlockSpec((_BLK, NW), lambda i, j: (i, 0)),
        ],
        out_specs=pl.BlockSpec((_BLK, HH), lambda i, j: (j * _NB + i, 0)),
        out_shape=jax.ShapeDtypeStruct((2 * N, HH), jnp.float32),
    )(x, w1, degt)


def _mm2_body(za_ref, zb_ref, wa_ref, wb_ref, ba_ref, bb_ref, degt_ref, y_ref):
    deg = jnp.sum(degt_ref[...], axis=1, keepdims=True) + 1.0
    dinv = lax.rsqrt(deg)
    ha = jnp.maximum(za_ref[...] * dinv + ba_ref[0], 0.0)
    hb = jnp.maximum(zb_ref[...] * dinv + bb_ref[0], 0.0)
    acc = jnp.dot(ha, wa_ref[...], preferred_element_type=jnp.float32)
    acc = acc + jnp.dot(hb, wb_ref[...], preferred_element_type=jnp.float32)
    y_ref[...] = acc * dinv


def _mm2(z1, w2, b1r, degt):
    return pl.pallas_call(
        _mm2_body,
        grid=(_NB, 2),
        in_specs=[
            pl.BlockSpec((_BLK, HH), lambda i, j: (i, 0)),
            pl.BlockSpec((_BLK, HH), lambda i, j: (_NB + i, 0)),
            pl.BlockSpec((HH, HH), lambda i, j: (0, j)),
            pl.BlockSpec((HH, HH), lambda i, j: (1, j)),
            pl.BlockSpec((1, 1, HH), lambda i, j: (0, 0, 0)),
            pl.BlockSpec((1, 1, HH), lambda i, j: (1, 0, 0)),
            pl.BlockSpec((_BLK, NW), lambda i, j: (i, 0)),
        ],
        out_specs=pl.BlockSpec((_BLK, HH), lambda i, j: (j * _NB + i, 0)),
        out_shape=jax.ShapeDtypeStruct((2 * N, HH), jnp.float32),
    )(z1, z1, w2, w2, b1r, b1r, degt)


def _mm3_body(za_ref, zb_ref, wa_ref, wb_ref, ba_ref, bb_ref, bl_ref, degt_ref,
              o_ref):
    deg = jnp.sum(degt_ref[...], axis=1, keepdims=True) + 1.0
    dinv = lax.rsqrt(deg)
    ha = jnp.maximum(za_ref[...] * dinv + ba_ref[0], 0.0)
    hb = jnp.maximum(zb_ref[...] * dinv + bb_ref[0], 0.0)
    acc = jnp.dot(ha, wa_ref[...], preferred_element_type=jnp.float32)
    acc = acc + jnp.dot(hb, wb_ref[...], preferred_element_type=jnp.float32)
    o_ref[...] = acc + bl_ref[...]


def _mm3(z2, wl, b2r, bl, degt):
    return pl.pallas_call(
        _mm3_body,
        grid=(_NB,),
        in_specs=[
            pl.BlockSpec((_BLK, HH), lambda i: (i, 0)),
            pl.BlockSpec((_BLK, HH), lambda i: (_NB + i, 0)),
            pl.BlockSpec((HH, 1), lambda i: (0, 0)),
            pl.BlockSpec((HH, 1), lambda i: (1, 0)),
            pl.BlockSpec((1, 1, HH), lambda i: (0, 0, 0)),
            pl.BlockSpec((1, 1, HH), lambda i: (1, 0, 0)),
            pl.BlockSpec((1, 1), lambda i: (0, 0)),
            pl.BlockSpec((_BLK, NW), lambda i: (i, 0)),
        ],
        out_specs=pl.BlockSpec((_BLK, 1), lambda i: (i, 0)),
        out_shape=jax.ShapeDtypeStruct((N, 1), jnp.float32),
    )(z2, z2, wl, wl, b2r, b2r, bl.reshape(1, 1), degt)


@jax.jit
def kernel(x, edge_index, batch, W1, b1, W2, b2, Wl, bl):
    ei3 = edge_index.reshape(2, NROW, CH)

    degp, eidx = _deg_kernel(ei3)
    degt = degp.T

    y1 = _mm1(x, W1, degt)
    z1 = _prop_kernel(y1, eidx)
    y2 = _mm2(z1, W2, b1.reshape(2, 1, HH), degt)
    z2 = _prop_kernel(y2, eidx)
    out = _mm3(z2, Wl, b2.reshape(2, 1, HH), bl, degt)
    return out


# speedup vs baseline: 1.0520x; 1.0156x over previous
import functools

import jax
import jax.numpy as jnp
from jax import lax
from jax.experimental import pallas as pl
from jax.experimental.pallas import tpu as pltpu
from jax.experimental.pallas import tpu_sc as plsc

N = 10000
E = 320000
F_IN = 128
H = 256
HH = H // 2

NC = 2
NS = 16
NW = NC * NS

CH = 80
RPT = N // NS

_sc_mesh = plsc.VectorSubcoreMesh(core_axis_name="c", subcore_axis_name="s")


NROW = E // CH
RPW = NROW // NW


@functools.partial(
    pl.kernel,
    mesh=_sc_mesh,
    out_type=[
        jax.ShapeDtypeStruct((NW, N), jnp.float32),
        jax.ShapeDtypeStruct((3, NROW, CH), jnp.int32),
    ],
    scratch_types=[
        pltpu.VMEM((N,), jnp.float32),
        pltpu.VMEM((RPW, CH), jnp.int32),
        pltpu.VMEM((RPW, CH), jnp.int32),
        pltpu.SemaphoreType.DMA,
        pltpu.SemaphoreType.DMA,
    ],
    compiler_params=pltpu.CompilerParams(
        needs_layout_passes=False, use_tc_tiling_on_sc=False
    ),
)
def _deg_kernel(ei_hbm, out_hbm, eidx_hbm, acc, dbuf, sbuf, dsem, ssem):
    c = lax.axis_index("c")
    s = lax.axis_index("s")
    wid = s * NC + c
    rlo = wid * RPW

    din = pltpu.async_copy(ei_hbm.at[1, pl.ds(rlo, RPW)], dbuf, dsem)
    sin = pltpu.async_copy(ei_hbm.at[0, pl.ds(rlo, RPW)], sbuf, ssem)

    z16 = jnp.zeros((16,), jnp.float32)

    def zinit(i, _):
        acc[pl.ds(i * 16, 16)] = z16
        return 0

    lax.fori_loop(0, N // 16, zinit, 0)
    din.wait()
    dout = pltpu.async_copy(dbuf, eidx_hbm.at[2, pl.ds(rlo, RPW)], dsem)
    sin.wait()
    sout = pltpu.async_copy(sbuf, eidx_hbm.at[0, pl.ds(rlo, RPW)], ssem)

    ones = jnp.full((16,), 1.0, jnp.float32)

    def body(i, _):
        for j in range(CH // 16):
            idx = dbuf[i, pl.ds(j * 16, 16)]
            plsc.addupdate_scatter(acc, [idx], ones)
        return 0

    lax.fori_loop(0, RPW, body, 0)
    pltpu.sync_copy(acc, out_hbm.at[wid])

    sout.wait()

    def addn(i, _):
        for j in range(CH // 16):
            sbuf[i, pl.ds(j * 16, 16)] = sbuf[i, pl.ds(j * 16, 16)] + N
        return 0

    lax.fori_loop(0, RPW, addn, 0)
    pltpu.sync_copy(sbuf, eidx_hbm.at[1, pl.ds(rlo, RPW)])
    dout.wait()


NCHUNK = E // NS // CH
NPH = 2
PCH = NCHUNK // NPH
NBUF = 3


@functools.partial(
    pl.kernel,
    mesh=_sc_mesh,
    out_type=jax.ShapeDtypeStruct((2 * N, HH), jnp.float32),
    scratch_types=[
        pltpu.VMEM((PCH, CH), jnp.int32),
        pltpu.VMEM((PCH, CH), jnp.int32),
        [pltpu.VMEM((CH, HH), jnp.float32)] * NBUF,
        [pltpu.SemaphoreType.DMA] * NBUF,
        [pltpu.SemaphoreType.DMA] * NBUF,
        pltpu.VMEM_SHARED((N, HH), jnp.float32),
        pltpu.SemaphoreType.DMA,
    ],
    compiler_params=pltpu.CompilerParams(use_tc_tiling_on_sc=False),
)
def _prop_kernel(y_hbm, eidx_hbm, z_hbm, sidx, didx, bufs, gsems,
                 ssems, zacc, isem):
    c = lax.axis_index("c")
    s = lax.axis_index("s")
    rbase = s * RPT

    pltpu.sync_copy(
        y_hbm.at[pl.ds(c * N + rbase, RPT)], zacc.at[pl.ds(rbase, RPT)]
    )
    plsc.subcore_barrier()

    def g_start(i, b):
        pltpu.async_copy(y_hbm.at[sidx.at[i]], bufs[b], gsems[b])

    def g_wait(b):
        pltpu.make_async_copy(y_hbm.at[sidx.at[0]], bufs[b], gsems[b]).wait()

    def s_start(i, b):
        pltpu.async_copy(bufs[b], zacc.at[didx.at[i]], ssems[b], add=True)

    def s_wait(b):
        pltpu.make_async_copy(bufs[b], zacc.at[didx.at[0]], ssems[b]).wait()

    def phase(p, _):
        roff = s * (NPH * PCH) + p * PCH
        iin = pltpu.async_copy(eidx_hbm.at[c, pl.ds(roff, PCH)], sidx, isem)
        pltpu.sync_copy(eidx_hbm.at[2, pl.ds(roff, PCH)], didx)
        iin.wait()
        for j in range(NBUF - 1):
            g_start(j, j)

        def step(i, _):
            for b in range(NBUF):
                @pl.when(lax.rem(i, NBUF) == b)
                def _(b=b):
                    nb = (b + NBUF - 1) % NBUF
                    g_wait(b)
                    s_start(i, b)

                    @pl.when(i >= 1)
                    def _():
                        s_wait(nb)

                    @pl.when(i + NBUF - 1 < PCH)
                    def _():
                        g_start(i + NBUF - 1, nb)

            return 0

        lax.fori_loop(0, PCH, step, 0)
        s_wait((PCH - 1) % NBUF)
        return 0

    lax.fori_loop(0, NPH, phase, 0)
    plsc.subcore_barrier()

    pltpu.sync_copy(
        zacc.at[pl.ds(rbase, RPT)], z_hbm.at[pl.ds(c * N + rbase, RPT)]
    )


def _mm1_body(x_ref, w_ref, degt_ref, y_ref):
    deg = jnp.sum(degt_ref[...], axis=1, keepdims=True) + 1.0
    dinv = lax.rsqrt(deg)
    xw = jnp.dot(x_ref[...], w_ref[...], preferred_element_type=jnp.float32)
    y_ref[...] = xw * dinv


_BLK = 5000
_NB = N // _BLK


def _mm1(x, w1, degt):
    return pl.pallas_call(
        _mm1_body,
        grid=(_NB, 2),
        in_specs=[
            pl.BlockSpec((_BLK, F_IN), lambda i, j: (i, 0)),
            pl.BlockSpec((F_IN, HH), lambda i, j: (0, j)),
            pl.BlockSpec((_BLK, NW), lambda i, j: (i, 0)),
        ],
        out_specs=pl.BlockSpec((_BLK, HH), lambda i, j: (j * _NB + i, 0)),
        out_shape=jax.ShapeDtypeStruct((2 * N, HH), jnp.float32),
    )(x, w1, degt)


def _mm2_body(za_ref, zb_ref, wa_ref, wb_ref, ba_ref, bb_ref, degt_ref, y_ref):
    deg = jnp.sum(degt_ref[...], axis=1, keepdims=True) + 1.0
    dinv = lax.rsqrt(deg)
    ha = jnp.maximum(za_ref[...] * dinv + ba_ref[0], 0.0)
    hb = jnp.maximum(zb_ref[...] * dinv + bb_ref[0], 0.0)
    acc = jnp.dot(ha, wa_ref[...], preferred_element_type=jnp.float32)
    acc = acc + jnp.dot(hb, wb_ref[...], preferred_element_type=jnp.float32)
    y_ref[...] = acc * dinv


def _mm2(z1, w2, b1r, degt):
    return pl.pallas_call(
        _mm2_body,
        grid=(_NB, 2),
        in_specs=[
            pl.BlockSpec((_BLK, HH), lambda i, j: (i, 0)),
            pl.BlockSpec((_BLK, HH), lambda i, j: (_NB + i, 0)),
            pl.BlockSpec((HH, HH), lambda i, j: (0, j)),
            pl.BlockSpec((HH, HH), lambda i, j: (1, j)),
            pl.BlockSpec((1, 1, HH), lambda i, j: (0, 0, 0)),
            pl.BlockSpec((1, 1, HH), lambda i, j: (1, 0, 0)),
            pl.BlockSpec((_BLK, NW), lambda i, j: (i, 0)),
        ],
        out_specs=pl.BlockSpec((_BLK, HH), lambda i, j: (j * _NB + i, 0)),
        out_shape=jax.ShapeDtypeStruct((2 * N, HH), jnp.float32),
    )(z1, z1, w2, w2, b1r, b1r, degt)


def _mm3_body(za_ref, zb_ref, wa_ref, wb_ref, ba_ref, bb_ref, bl_ref, degt_ref,
              o_ref):
    deg = jnp.sum(degt_ref[...], axis=1, keepdims=True) + 1.0
    dinv = lax.rsqrt(deg)
    ha = jnp.maximum(za_ref[...] * dinv + ba_ref[0], 0.0)
    hb = jnp.maximum(zb_ref[...] * dinv + bb_ref[0], 0.0)
    acc = jnp.dot(ha, wa_ref[...], preferred_element_type=jnp.float32)
    acc = acc + jnp.dot(hb, wb_ref[...], preferred_element_type=jnp.float32)
    o_ref[...] = acc + bl_ref[...]


def _mm3(z2, wl, b2r, bl, degt):
    return pl.pallas_call(
        _mm3_body,
        grid=(_NB,),
        in_specs=[
            pl.BlockSpec((_BLK, HH), lambda i: (i, 0)),
            pl.BlockSpec((_BLK, HH), lambda i: (_NB + i, 0)),
            pl.BlockSpec((HH, 1), lambda i: (0, 0)),
            pl.BlockSpec((HH, 1), lambda i: (1, 0)),
            pl.BlockSpec((1, 1, HH), lambda i: (0, 0, 0)),
            pl.BlockSpec((1, 1, HH), lambda i: (1, 0, 0)),
            pl.BlockSpec((1, 1), lambda i: (0, 0)),
            pl.BlockSpec((_BLK, NW), lambda i: (i, 0)),
        ],
        out_specs=pl.BlockSpec((_BLK, 1), lambda i: (i, 0)),
        out_shape=jax.ShapeDtypeStruct((N, 1), jnp.float32),
    )(z2, z2, wl, wl, b2r, b2r, bl.reshape(1, 1), degt)


@jax.jit
def kernel(x, edge_index, batch, W1, b1, W2, b2, Wl, bl):

    ei3 = edge_index.reshape(2, NROW, CH)

    degp, eidx = _deg_kernel(ei3)
    degt = degp.T

    y1 = _mm1(x, W1, degt)
    z1 = _prop_kernel(y1, eidx)
    y2 = _mm2(z1, W2, b1.reshape(2, 1, HH), degt)
    z2 = _prop_kernel(y2, eidx)
    out = _mm3(z2, Wl, b2.reshape(2, 1, HH), bl, degt)
    return out


# speedup vs baseline: 1.0596x; 1.0073x over previous
import functools

import jax
import jax.numpy as jnp
from jax import lax
from jax.experimental import pallas as pl
from jax.experimental.pallas import tpu as pltpu
from jax.experimental.pallas import tpu_sc as plsc

N = 10000
E = 320000
F_IN = 128
H = 256
HH = H // 2

NC = 2
NS = 16
NW = NC * NS

CH = 80
RPT = N // NS

_sc_mesh = plsc.VectorSubcoreMesh(core_axis_name="c", subcore_axis_name="s")


NROW = E // CH
RPW = NROW // NW


@functools.partial(
    pl.kernel,
    mesh=_sc_mesh,
    out_type=[
        jax.ShapeDtypeStruct((NW, N), jnp.float32),
        jax.ShapeDtypeStruct((3, NROW, CH), jnp.int32),
    ],
    scratch_types=[
        pltpu.VMEM((N,), jnp.float32),
        pltpu.VMEM((RPW, CH), jnp.int32),
        pltpu.VMEM((RPW, CH), jnp.int32),
        pltpu.SemaphoreType.DMA,
        pltpu.SemaphoreType.DMA,
    ],
    compiler_params=pltpu.CompilerParams(
        needs_layout_passes=False, use_tc_tiling_on_sc=False
    ),
)
def _deg_kernel(ei_hbm, out_hbm, eidx_hbm, acc, dbuf, sbuf, dsem, ssem):
    c = lax.axis_index("c")
    s = lax.axis_index("s")
    wid = s * NC + c
    rlo = wid * RPW

    din = pltpu.async_copy(ei_hbm.at[1, pl.ds(rlo, RPW)], dbuf, dsem)
    sin = pltpu.async_copy(ei_hbm.at[0, pl.ds(rlo, RPW)], sbuf, ssem)

    z16 = jnp.zeros((16,), jnp.float32)

    def zinit(i, _):
        acc[pl.ds(i * 16, 16)] = z16
        return 0

    lax.fori_loop(0, N // 16, zinit, 0)
    din.wait()
    dout = pltpu.async_copy(dbuf, eidx_hbm.at[2, pl.ds(rlo, RPW)], dsem)
    sin.wait()
    sout = pltpu.async_copy(sbuf, eidx_hbm.at[0, pl.ds(rlo, RPW)], ssem)

    ones = jnp.full((16,), 1.0, jnp.float32)

    def body(i, _):
        for j in range(CH // 16):
            idx = dbuf[i, pl.ds(j * 16, 16)]
            plsc.addupdate_scatter(acc, [idx], ones)
        return 0

    lax.fori_loop(0, RPW, body, 0)
    pltpu.sync_copy(acc, out_hbm.at[wid])

    sout.wait()

    def addn(i, _):
        for j in range(CH // 16):
            sbuf[i, pl.ds(j * 16, 16)] = sbuf[i, pl.ds(j * 16, 16)] + N
        return 0

    lax.fori_loop(0, RPW, addn, 0)
    pltpu.sync_copy(sbuf, eidx_hbm.at[1, pl.ds(rlo, RPW)])
    dout.wait()


NCHUNK = E // NS // CH
NPH = 2
PCH = NCHUNK // NPH
NBUF = 3


@functools.partial(
    pl.kernel,
    mesh=_sc_mesh,
    out_type=jax.ShapeDtypeStruct((2 * N, HH), jnp.float32),
    scratch_types=[
        pltpu.VMEM((PCH, CH), jnp.int32),
        pltpu.VMEM((PCH, CH), jnp.int32),
        [pltpu.VMEM((CH, HH), jnp.float32)] * NBUF,
        [pltpu.SemaphoreType.DMA] * NBUF,
        [pltpu.SemaphoreType.DMA] * NBUF,
        pltpu.VMEM_SHARED((N, HH), jnp.float32),
        pltpu.SemaphoreType.DMA,
    ],
    compiler_params=pltpu.CompilerParams(use_tc_tiling_on_sc=False),
)
def _prop_kernel(y_hbm, eidx_hbm, z_hbm, sidx, didx, bufs, gsems,
                 ssems, zacc, isem):
    c = lax.axis_index("c")
    s = lax.axis_index("s")
    rbase = s * RPT

    pltpu.sync_copy(
        y_hbm.at[pl.ds(c * N + rbase, RPT)], zacc.at[pl.ds(rbase, RPT)]
    )
    plsc.subcore_barrier()

    def g_start(i, b):
        pltpu.async_copy(y_hbm.at[sidx.at[i]], bufs[b], gsems[b])

    def g_wait(b):
        pltpu.make_async_copy(y_hbm.at[sidx.at[0]], bufs[b], gsems[b]).wait()

    def s_start(i, b):
        pltpu.async_copy(bufs[b], zacc.at[didx.at[i]], ssems[b], add=True)

    def s_wait(b):
        pltpu.make_async_copy(bufs[b], zacc.at[didx.at[0]], ssems[b]).wait()

    def phase(p, _):
        roff = s * (NPH * PCH) + p * PCH
        iin = pltpu.async_copy(eidx_hbm.at[c, pl.ds(roff, PCH)], sidx, isem)
        pltpu.sync_copy(eidx_hbm.at[2, pl.ds(roff, PCH)], didx)
        iin.wait()
        for j in range(NBUF - 1):
            g_start(j, j)

        def step(i, _):
            for b in range(NBUF):
                @pl.when(lax.rem(i, NBUF) == b)
                def _(b=b):
                    nb = (b + NBUF - 1) % NBUF
                    g_wait(b)
                    s_start(i, b)

                    @pl.when(i >= 1)
                    def _():
                        s_wait(nb)

                    @pl.when(i + NBUF - 1 < PCH)
                    def _():
                        g_start(i + NBUF - 1, nb)

            return 0

        lax.fori_loop(0, PCH, step, 0)
        s_wait((PCH - 1) % NBUF)
        return 0

    lax.fori_loop(0, NPH, phase, 0)
    plsc.subcore_barrier()

    pltpu.sync_copy(
        zacc.at[pl.ds(rbase, RPT)], z_hbm.at[pl.ds(c * N + rbase, RPT)]
    )


def _mm1_body(x_ref, w_ref, degt_ref, y_ref):
    deg = jnp.sum(degt_ref[...], axis=1, keepdims=True) + 1.0
    dinv = lax.rsqrt(deg)
    xw = jnp.dot(x_ref[...], w_ref[...], preferred_element_type=jnp.float32)
    y_ref[...] = xw * dinv


_BLK = 10000
_NB = N // _BLK


def _mm1(x, w1, degt):
    return pl.pallas_call(
        _mm1_body,
        grid=(_NB, 2),
        in_specs=[
            pl.BlockSpec((_BLK, F_IN), lambda i, j: (i, 0)),
            pl.BlockSpec((F_IN, HH), lambda i, j: (0, j)),
            pl.BlockSpec((_BLK, NW), lambda i, j: (i, 0)),
        ],
        out_specs=pl.BlockSpec((_BLK, HH), lambda i, j: (j * _NB + i, 0)),
        out_shape=jax.ShapeDtypeStruct((2 * N, HH), jnp.float32),
    )(x, w1, degt)


def _mm2_body(za_ref, zb_ref, wa_ref, wb_ref, ba_ref, bb_ref, degt_ref, y_ref):
    deg = jnp.sum(degt_ref[...], axis=1, keepdims=True) + 1.0
    dinv = lax.rsqrt(deg)
    ha = jnp.maximum(za_ref[...] * dinv + ba_ref[0], 0.0)
    hb = jnp.maximum(zb_ref[...] * dinv + bb_ref[0], 0.0)
    acc = jnp.dot(ha, wa_ref[...], preferred_element_type=jnp.float32)
    acc = acc + jnp.dot(hb, wb_ref[...], preferred_element_type=jnp.float32)
    y_ref[...] = acc * dinv


def _mm2(z1, w2, b1r, degt):
    return pl.pallas_call(
        _mm2_body,
        grid=(_NB, 2),
        in_specs=[
            pl.BlockSpec((_BLK, HH), lambda i, j: (i, 0)),
            pl.BlockSpec((_BLK, HH), lambda i, j: (_NB + i, 0)),
            pl.BlockSpec((HH, HH), lambda i, j: (0, j)),
            pl.BlockSpec((HH, HH), lambda i, j: (1, j)),
            pl.BlockSpec((1, 1, HH), lambda i, j: (0, 0, 0)),
            pl.BlockSpec((1, 1, HH), lambda i, j: (1, 0, 0)),
            pl.BlockSpec((_BLK, NW), lambda i, j: (i, 0)),
        ],
        out_specs=pl.BlockSpec((_BLK, HH), lambda i, j: (j * _NB + i, 0)),
        out_shape=jax.ShapeDtypeStruct((2 * N, HH), jnp.float32),
    )(z1, z1, w2, w2, b1r, b1r, degt)


def _mm3_body(za_ref, zb_ref, wa_ref, wb_ref, ba_ref, bb_ref, bl_ref, degt_ref,
              o_ref):
    deg = jnp.sum(degt_ref[...], axis=1, keepdims=True) + 1.0
    dinv = lax.rsqrt(deg)
    ha = jnp.maximum(za_ref[...] * dinv + ba_ref[0], 0.0)
    hb = jnp.maximum(zb_ref[...] * dinv + bb_ref[0], 0.0)
    acc = jnp.dot(ha, wa_ref[...], preferred_element_type=jnp.float32)
    acc = acc + jnp.dot(hb, wb_ref[...], preferred_element_type=jnp.float32)
    o_ref[...] = acc + bl_ref[...]


def _mm3(z2, wl, b2r, bl, degt):
    return pl.pallas_call(
        _mm3_body,
        grid=(_NB,),
        in_specs=[
            pl.BlockSpec((_BLK, HH), lambda i: (i, 0)),
            pl.BlockSpec((_BLK, HH), lambda i: (_NB + i, 0)),
            pl.BlockSpec((HH, 1), lambda i: (0, 0)),
            pl.BlockSpec((HH, 1), lambda i: (1, 0)),
            pl.BlockSpec((1, 1, HH), lambda i: (0, 0, 0)),
            pl.BlockSpec((1, 1, HH), lambda i: (1, 0, 0)),
            pl.BlockSpec((1, 1), lambda i: (0, 0)),
            pl.BlockSpec((_BLK, NW), lambda i: (i, 0)),
        ],
        out_specs=pl.BlockSpec((_BLK, 1), lambda i: (i, 0)),
        out_shape=jax.ShapeDtypeStruct((N, 1), jnp.float32),
    )(z2, z2, wl, wl, b2r, b2r, bl.reshape(1, 1), degt)


@jax.jit
def kernel(x, edge_index, batch, W1, b1, W2, b2, Wl, bl):

    ei3 = edge_index.reshape(2, NROW, CH)

    degp, eidx = _deg_kernel(ei3)
    degt = degp.T

    y1 = _mm1(x, W1, degt)
    z1 = _prop_kernel(y1, eidx)
    y2 = _mm2(z1, W2, b1.reshape(2, 1, HH), degt)
    z2 = _prop_kernel(y2, eidx)
    out = _mm3(z2, Wl, b2.reshape(2, 1, HH), bl, degt)
    return out
